# baseline (speedup 1.0000x reference)
import functools

import jax
import jax.numpy as jnp
from jax import lax
from jax.experimental import pallas as pl
from jax.experimental.pallas import tpu as pltpu
from jax.experimental.pallas import tpu_sc as plsc

N = 50000
CIN = 64
COUT = 64
W = 72
KK = 32
S = 32
NBINS = 32768
HALF = NBINS // 2
PADN = 51200
TN = 512
PPT = PADN // 16
TAIL = PADN // 32

_I16 = lambda: lax.iota(jnp.int32, 16)


def _popcount16(occb):
    v = jnp.where(occb, 1, 0)
    s = jnp.int32(0)
    for r in range(16):
        s = s + v[r]
    return s


def _mm_body(coords_ref, feats_ref, kflat_ref, out_ref, bin_ref):
    n = pl.program_id(0)
    c1 = coords_ref[:, 1:2]
    f = feats_ref[...]
    kb = kflat_ref[...].astype(jnp.bfloat16)
    parts = [jnp.where(c1 == k, f, 0.0) for k in range(KK)]
    stacked = jnp.concatenate(parts, axis=1)
    acc = jnp.dot(stacked.astype(jnp.bfloat16), kb,
                  preferred_element_type=jnp.float32)
    out_ref[...] = jnp.concatenate(
        [acc, jnp.ones((TN, W - COUT), jnp.float32)], axis=1)
    rowid = n * TN + lax.broadcasted_iota(jnp.int32, (TN, 1), 0)
    b = (coords_ref[:, 0:1] * 1024 + coords_ref[:, 2:3] * 32
         + coords_ref[:, 3:4])
    bin_ref[...] = jnp.where(rowid < N, b, NBINS).reshape(1, TN // 128, 128)


def _run_matmul(coords, feats, kflat):
    nlast = (N - 1) // TN

    def _in_map(n):
        return (jnp.minimum(n, nlast), 0)

    return pl.pallas_call(
        _mm_body,
        grid=(PADN // TN,),
        in_specs=[
            pl.BlockSpec((TN, 4), _in_map),
            pl.BlockSpec((TN, CIN), _in_map),
            pl.BlockSpec((KK * CIN, COUT), lambda n: (0, 0)),
        ],
        out_specs=[
            pl.BlockSpec((TN, W), lambda n: (n, 0)),
            pl.BlockSpec((1, TN // 128, 128), lambda n: (n, 0, 0)),
        ],
        out_shape=[
            jax.ShapeDtypeStruct((PADN, W), jnp.float32),
            jax.ShapeDtypeStruct((PADN // TN, TN // 128, 128), jnp.int32),
        ],
    )(coords, feats, kflat)


def _pf_body(cnt_ref, pos_ref, mtot_ref):
    if True:
        occ = (cnt_ref[...] > 0).astype(jnp.float32)
        ir = lax.broadcasted_iota(jnp.int32, (128, 128), 0)
        ic = lax.broadcasted_iota(jnp.int32, (128, 128), 1)
        slt = (ir < ic).astype(jnp.float32)
        ex_row = jnp.dot(occ, slt, preferred_element_type=jnp.float32)
        row_tot = jnp.dot(occ, jnp.ones((128, 1), jnp.float32),
                          preferred_element_type=jnp.float32)
        jr = lax.broadcasted_iota(jnp.int32, (256, 256), 0)
        jc = lax.broadcasted_iota(jnp.int32, (256, 256), 1)
        slt2 = (jc < jr).astype(jnp.float32)
        row_off = jnp.dot(slt2, row_tot, preferred_element_type=jnp.float32)
        posf = row_off + ex_row
        br = lax.broadcasted_iota(jnp.int32, (256, 128), 0)
        bc = lax.broadcasted_iota(jnp.int32, (256, 128), 1)
        trash = PADN + ((br * 128 + bc) & 8191)
        pos_ref[...] = jnp.where(occ > 0, posf.astype(jnp.int32), trash)
        total = jnp.sum(row_tot).astype(jnp.int32)
        mtot_ref[...] = jnp.full((8, 128), total, jnp.int32)


def _run_prefix(cnt2d):
    return pl.pallas_call(
        _pf_body,
        out_shape=[
            jax.ShapeDtypeStruct((256, 128), jnp.int32),
            jax.ShapeDtypeStruct((8, 128), jnp.int32),
        ],
    )(cnt2d)


_MESH = plsc.VectorSubcoreMesh(core_axis_name="c", subcore_axis_name="s")


@functools.partial(
    pl.kernel,
    out_type=(
        jax.ShapeDtypeStruct((NBINS + 1, W), jnp.float32),
        jax.ShapeDtypeStruct((NBINS,), jnp.int32),
    ),
    mesh=_MESH,
    compiler_params=pltpu.CompilerParams(use_tc_tiling_on_sc=False),
    scratch_types=(
        pltpu.VMEM((2, 128, W), jnp.float32),
        pltpu.VMEM((PPT,), jnp.int32),
        pltpu.VMEM((2, 128), jnp.int32),
        pltpu.VMEM((1024,), jnp.int32),
        pltpu.VMEM_SHARED((HALF + 1, W), jnp.float32),
        pltpu.SemaphoreType.DMA,
        pltpu.SemaphoreType.DMA,
    ),
)
def _scatter_kernel(out72_hbm, bid_hbm, acc_hbm, cnt_hbm,
                    rows_v, bid_v, idx_v, c1d_v, acc_sh, sem_in, sem_sc):
    c = lax.axis_index("c")
    s = lax.axis_index("s")
    zf = jnp.zeros((16,), jnp.float32)

    def _zero_rows(i, _):
        for b in range(2):
            for cw in range(4):
                rows_v[b, i, pl.ds(cw * 16, 16)] = zf
            rows_v[b, i, pl.ds(W - 16, 16)] = zf
        return 0

    lax.fori_loop(0, 128, _zero_rows, 0)

    zdescs = [
        pltpu.async_copy(rows_v.at[h % 2],
                         acc_sh.at[pl.ds(s * 1024 + h * 128, 128)], sem_in)
        for h in range(8)
    ]

    @pl.when(s == 0)
    def _():
        pltpu.sync_copy(rows_v.at[0, pl.ds(0, 1)], acc_sh.at[pl.ds(HALF, 1)])

    for d in zdescs:
        d.wait()
    plsc.subcore_barrier()

    base = s * PPT
    nch = PPT // 128
    pltpu.sync_copy(bid_hbm.at[pl.ds(base, PPT)], bid_v)
    in_descs = [None] * nch
    sc_descs = [None] * nch
    in_descs[0] = pltpu.async_copy(out72_hbm.at[pl.ds(base, 128)],
                                   rows_v.at[0], sem_in)
    for ch in range(nch):
        b = ch % 2
        in_descs[ch].wait()
        if ch >= 1:
            sc_descs[ch - 1].wait()
        if ch + 1 < nch:
            in_descs[ch + 1] = pltpu.async_copy(
                out72_hbm.at[pl.ds(base + (ch + 1) * 128, 128)],
                rows_v.at[1 - b], sem_in)

        def _mk_idx(i, _, ch=ch, b=b):
            bd = bid_v[pl.ds(ch * 128 + i * 16, 16)]
            loc = bd - c * HALF
            ok = (loc >= 0) & (loc < HALF)
            loc = jnp.where(ok, loc, HALF)
            idx_v[b, pl.ds(i * 16, 16)] = loc
            return 0

        lax.fori_loop(0, 8, _mk_idx, 0)
        sc_descs[ch] = pltpu.async_copy(rows_v.at[b], acc_sh.at[idx_v.at[b]],
                                        sem_sc, add=True)
    sc_descs[nch - 1].wait()

    plsc.subcore_barrier()

    g = c * 16 + s
    gbase = g * 1024
    lane = _I16()
    wb_in = [None] * 8
    wb_out = [None] * 8
    wb_in[0] = pltpu.async_copy(acc_sh.at[pl.ds(s * 1024, 128)],
                                rows_v.at[0], sem_in)
    for h in range(8):
        b = h % 2
        wb_in[h].wait()
        if h >= 1:
            wb_out[h - 1].wait()
        if h + 1 < 8:
            wb_in[h + 1] = pltpu.async_copy(
                acc_sh.at[pl.ds(s * 1024 + (h + 1) * 128, 128)],
                rows_v.at[1 - b], sem_in)
        wb_out[h] = pltpu.async_copy(
            rows_v.at[b], acc_hbm.at[pl.ds(gbase + h * 128, 128)], sem_sc)

        def _extract(w, _, h=h, b=b):
            cv = jnp.zeros((16,), jnp.float32)
            for r in range(16):
                cnt_r = rows_v[b, w * 16 + r, pl.ds(W - 16, 16)][8]
                cv = jnp.where(lane == r, cnt_r, cv)
            c1d_v[pl.ds(h * 128 + w * 16, 16)] = cv.astype(jnp.int32)
            return 0

        lax.fori_loop(0, 8, _extract, 0)
    wb_out[7].wait()
    pltpu.sync_copy(c1d_v, cnt_hbm.at[pl.ds(gbase, 1024)])


OUTR = PADN + 8192


@functools.partial(
    pl.kernel,
    out_type=(
        jax.ShapeDtypeStruct((OUTR, W), jnp.float32),
        jax.ShapeDtypeStruct((OUTR, 16), jnp.int32),
    ),
    mesh=_MESH,
    compiler_params=pltpu.CompilerParams(use_tc_tiling_on_sc=False),
    scratch_types=(
        pltpu.VMEM((8, 128), jnp.int32),
        pltpu.VMEM((2, 128, W), jnp.float32),
        pltpu.VMEM((1024, 16), jnp.int32),
        pltpu.SemaphoreType.DMA,
        pltpu.SemaphoreType.DMA,
    ),
)
def _compact_kernel(acc_hbm, pos_hbm, dec_hbm, out_hbm, idx_hbm,
                    pos8_v, rows_v, dec_v, sem_in, sem_out):
    c = lax.axis_index("c")
    s = lax.axis_index("s")
    g = c * 16 + s

    d_pos = pltpu.async_copy(pos_hbm.at[pl.ds(g * 8, 8)], pos8_v, sem_in)
    d_dec = pltpu.async_copy(dec_hbm.at[pl.ds(g * 1024, 1024)], dec_v, sem_in)
    d_pos.wait()
    d_dec.wait()

    in_d = [None] * 8
    out_d = [None] * 8
    idx_d = [None] * 8
    in_d[0] = pltpu.async_copy(acc_hbm.at[pl.ds(g * 1024, 128)],
                               rows_v.at[0], sem_in)
    for j in range(8):
        b = j % 2
        in_d[j].wait()
        if j >= 1:
            out_d[j - 1].wait()
            idx_d[j - 1].wait()
        if j + 1 < 8:
            in_d[j + 1] = pltpu.async_copy(
                acc_hbm.at[pl.ds(g * 1024 + (j + 1) * 128, 128)],
                rows_v.at[1 - b], sem_in)
        out_d[j] = pltpu.async_copy(rows_v.at[b], out_hbm.at[pos8_v.at[j]],
                                    sem_out)
        idx_d[j] = pltpu.async_copy(dec_v.at[pl.ds(j * 128, 128)],
                                    idx_hbm.at[pos8_v.at[j]], sem_out)
    out_d[7].wait()
    idx_d[7].wait()


def kernel(feats, coords, kernel):
    kmat = kernel
    out72, bin2d = _run_matmul(coords.astype(jnp.int32), feats,
                               kmat.reshape(KK * CIN, COUT))
    binid = bin2d.reshape(PADN)

    acc, cnt = _scatter_kernel(out72, binid)
    pos2d, mtot = _run_prefix(cnt.reshape(256, 128))
    ar = jnp.arange(NBINS, dtype=jnp.int32)[:, None]
    cid = jnp.arange(16, dtype=jnp.int32)[None, :]
    dec = jnp.where(cid == 0, ar // 1024,
                    jnp.where(cid == 2, (ar // 32) % 32,
                              jnp.where(cid == 3, ar % 32, 0)))
    vals, idx16 = _compact_kernel(acc, pos2d, dec)

    total = mtot[0, 0]
    rid = jnp.arange(N, dtype=jnp.int32)[:, None]
    vals_out = jnp.where(rid < total, vals[:N, :COUT], 0.0)
    pad_row = jnp.array([-1, 31, 31, 31], jnp.int32)
    idx_out = jnp.where(rid < total, idx16[:N, :4], pad_row[None, :])
    return vals_out, idx_out


# speedup vs baseline: 1.0154x; 1.0154x over previous
import functools

import jax
import jax.numpy as jnp
from jax import lax
from jax.experimental import pallas as pl
from jax.experimental.pallas import tpu as pltpu
from jax.experimental.pallas import tpu_sc as plsc

N = 50000
CIN = 64
COUT = 64
W = 72
KK = 32
S = 32
NBINS = 32768
HALF = NBINS // 2
PADN = 51200
TN = 512
PPT = PADN // 16
TAIL = PADN // 32

_I16 = lambda: lax.iota(jnp.int32, 16)


def _popcount16(occb):
    v = jnp.where(occb, 1, 0)
    s = jnp.int32(0)
    for r in range(16):
        s = s + v[r]
    return s


def _mm_body(coords_ref, feats_ref, kflat_ref, out_ref, bin_ref):
    n = pl.program_id(0)
    c1 = coords_ref[:, 1:2]
    f = feats_ref[...].astype(jnp.bfloat16)
    kb = kflat_ref[...].astype(jnp.bfloat16)
    zero = jnp.zeros((TN, CIN), jnp.bfloat16)
    parts = [jnp.where(c1 == k, f, zero) for k in range(KK)]
    stacked = jnp.concatenate(parts, axis=1)
    acc = jnp.dot(stacked, kb, preferred_element_type=jnp.float32)
    out_ref[...] = jnp.concatenate(
        [acc, jnp.ones((TN, W - COUT), jnp.float32)], axis=1)
    rowid = n * TN + lax.broadcasted_iota(jnp.int32, (TN, 1), 0)
    b = (coords_ref[:, 0:1] * 1024 + coords_ref[:, 2:3] * 32
         + coords_ref[:, 3:4])
    bin_ref[...] = jnp.where(rowid < N, b, NBINS).reshape(1, TN // 128, 128)


def _run_matmul(coords, feats, kflat):
    nlast = (N - 1) // TN

    def _in_map(n):
        return (jnp.minimum(n, nlast), 0)

    return pl.pallas_call(
        _mm_body,
        grid=(PADN // TN,),
        in_specs=[
            pl.BlockSpec((TN, 4), _in_map),
            pl.BlockSpec((TN, CIN), _in_map),
            pl.BlockSpec((KK * CIN, COUT), lambda n: (0, 0)),
        ],
        out_specs=[
            pl.BlockSpec((TN, W), lambda n: (n, 0)),
            pl.BlockSpec((1, TN // 128, 128), lambda n: (n, 0, 0)),
        ],
        out_shape=[
            jax.ShapeDtypeStruct((PADN, W), jnp.float32),
            jax.ShapeDtypeStruct((PADN // TN, TN // 128, 128), jnp.int32),
        ],
    )(coords, feats, kflat)


def _pf_body(cnt_ref, pos_ref, mtot_ref):
    if True:
        occ = (cnt_ref[...] > 0).astype(jnp.float32)
        ir = lax.broadcasted_iota(jnp.int32, (128, 128), 0)
        ic = lax.broadcasted_iota(jnp.int32, (128, 128), 1)
        slt = (ir < ic).astype(jnp.float32)
        ex_row = jnp.dot(occ, slt, preferred_element_type=jnp.float32)
        row_tot = jnp.dot(occ, jnp.ones((128, 1), jnp.float32),
                          preferred_element_type=jnp.float32)
        jr = lax.broadcasted_iota(jnp.int32, (256, 256), 0)
        jc = lax.broadcasted_iota(jnp.int32, (256, 256), 1)
        slt2 = (jc < jr).astype(jnp.float32)
        row_off = jnp.dot(slt2, row_tot, preferred_element_type=jnp.float32)
        posf = row_off + ex_row
        br = lax.broadcasted_iota(jnp.int32, (256, 128), 0)
        bc = lax.broadcasted_iota(jnp.int32, (256, 128), 1)
        trash = PADN + ((br * 128 + bc) & 8191)
        pos_ref[...] = jnp.where(occ > 0, posf.astype(jnp.int32), trash)
        total = jnp.sum(row_tot).astype(jnp.int32)
        mtot_ref[...] = jnp.full((8, 128), total, jnp.int32)


def _run_prefix(cnt2d):
    return pl.pallas_call(
        _pf_body,
        out_shape=[
            jax.ShapeDtypeStruct((256, 128), jnp.int32),
            jax.ShapeDtypeStruct((8, 128), jnp.int32),
        ],
    )(cnt2d)


_MESH = plsc.VectorSubcoreMesh(core_axis_name="c", subcore_axis_name="s")


@functools.partial(
    pl.kernel,
    out_type=(
        jax.ShapeDtypeStruct((NBINS + 1, W), jnp.float32),
        jax.ShapeDtypeStruct((NBINS,), jnp.int32),
    ),
    mesh=_MESH,
    compiler_params=pltpu.CompilerParams(use_tc_tiling_on_sc=False),
    scratch_types=(
        pltpu.VMEM((2, 128, W), jnp.float32),
        pltpu.VMEM((PPT,), jnp.int32),
        pltpu.VMEM((2, 128), jnp.int32),
        pltpu.VMEM((1024,), jnp.int32),
        pltpu.VMEM_SHARED((HALF + 1, W), jnp.float32),
        pltpu.SemaphoreType.DMA,
        pltpu.SemaphoreType.DMA,
    ),
)
def _scatter_kernel(out72_hbm, bid_hbm, acc_hbm, cnt_hbm,
                    rows_v, bid_v, idx_v, c1d_v, acc_sh, sem_in, sem_sc):
    c = lax.axis_index("c")
    s = lax.axis_index("s")
    zf = jnp.zeros((16,), jnp.float32)

    def _zero_rows(i, _):
        for b in range(2):
            for cw in range(4):
                rows_v[b, i, pl.ds(cw * 16, 16)] = zf
            rows_v[b, i, pl.ds(W - 16, 16)] = zf
        return 0

    lax.fori_loop(0, 128, _zero_rows, 0)

    zdescs = [
        pltpu.async_copy(rows_v.at[h % 2],
                         acc_sh.at[pl.ds(s * 1024 + h * 128, 128)], sem_in)
        for h in range(8)
    ]

    @pl.when(s == 0)
    def _():
        pltpu.sync_copy(rows_v.at[0, pl.ds(0, 1)], acc_sh.at[pl.ds(HALF, 1)])

    for d in zdescs:
        d.wait()
    plsc.subcore_barrier()

    base = s * PPT
    nch = PPT // 128
    pltpu.sync_copy(bid_hbm.at[pl.ds(base, PPT)], bid_v)
    in_descs = [None] * nch
    sc_descs = [None] * nch
    in_descs[0] = pltpu.async_copy(out72_hbm.at[pl.ds(base, 128)],
                                   rows_v.at[0], sem_in)
    for ch in range(nch):
        b = ch % 2
        in_descs[ch].wait()
        if ch >= 1:
            sc_descs[ch - 1].wait()
        if ch + 1 < nch:
            in_descs[ch + 1] = pltpu.async_copy(
                out72_hbm.at[pl.ds(base + (ch + 1) * 128, 128)],
                rows_v.at[1 - b], sem_in)

        def _mk_idx(i, _, ch=ch, b=b):
            bd = bid_v[pl.ds(ch * 128 + i * 16, 16)]
            loc = bd - c * HALF
            ok = (loc >= 0) & (loc < HALF)
            loc = jnp.where(ok, loc, HALF)
            idx_v[b, pl.ds(i * 16, 16)] = loc
            return 0

        lax.fori_loop(0, 8, _mk_idx, 0)
        sc_descs[ch] = pltpu.async_copy(rows_v.at[b], acc_sh.at[idx_v.at[b]],
                                        sem_sc, add=True)
    sc_descs[nch - 1].wait()

    plsc.subcore_barrier()

    g = c * 16 + s
    gbase = g * 1024
    lane = _I16()
    wb_in = [None] * 8
    wb_out = [None] * 8
    wb_in[0] = pltpu.async_copy(acc_sh.at[pl.ds(s * 1024, 128)],
                                rows_v.at[0], sem_in)
    for h in range(8):
        b = h % 2
        wb_in[h].wait()
        if h >= 1:
            wb_out[h - 1].wait()
        if h + 1 < 8:
            wb_in[h + 1] = pltpu.async_copy(
                acc_sh.at[pl.ds(s * 1024 + (h + 1) * 128, 128)],
                rows_v.at[1 - b], sem_in)
        wb_out[h] = pltpu.async_copy(
            rows_v.at[b], acc_hbm.at[pl.ds(gbase + h * 128, 128)], sem_sc)

        def _extract(w, _, h=h, b=b):
            cv = jnp.zeros((16,), jnp.float32)
            for r in range(16):
                cnt_r = rows_v[b, w * 16 + r, pl.ds(W - 16, 16)][8]
                cv = jnp.where(lane == r, cnt_r, cv)
            c1d_v[pl.ds(h * 128 + w * 16, 16)] = cv.astype(jnp.int32)
            return 0

        lax.fori_loop(0, 8, _extract, 0)
    wb_out[7].wait()
    pltpu.sync_copy(c1d_v, cnt_hbm.at[pl.ds(gbase, 1024)])


OUTR = PADN + 8192


@functools.partial(
    pl.kernel,
    out_type=(
        jax.ShapeDtypeStruct((OUTR, W), jnp.float32),
        jax.ShapeDtypeStruct((OUTR, 16), jnp.int32),
    ),
    mesh=_MESH,
    compiler_params=pltpu.CompilerParams(use_tc_tiling_on_sc=False),
    scratch_types=(
        pltpu.VMEM((8, 128), jnp.int32),
        pltpu.VMEM((2, 128, W), jnp.float32),
        pltpu.VMEM((1024, 16), jnp.int32),
        pltpu.SemaphoreType.DMA,
        pltpu.SemaphoreType.DMA,
    ),
)
def _compact_kernel(acc_hbm, pos_hbm, dec_hbm, out_hbm, idx_hbm,
                    pos8_v, rows_v, dec_v, sem_in, sem_out):
    c = lax.axis_index("c")
    s = lax.axis_index("s")
    g = c * 16 + s

    d_pos = pltpu.async_copy(pos_hbm.at[pl.ds(g * 8, 8)], pos8_v, sem_in)
    d_dec = pltpu.async_copy(dec_hbm.at[pl.ds(g * 1024, 1024)], dec_v, sem_in)
    d_pos.wait()
    d_dec.wait()

    in_d = [None] * 8
    out_d = [None] * 8
    idx_d = [None] * 8
    in_d[0] = pltpu.async_copy(acc_hbm.at[pl.ds(g * 1024, 128)],
                               rows_v.at[0], sem_in)
    for j in range(8):
        b = j % 2
        in_d[j].wait()
        if j >= 1:
            out_d[j - 1].wait()
            idx_d[j - 1].wait()
        if j + 1 < 8:
            in_d[j + 1] = pltpu.async_copy(
                acc_hbm.at[pl.ds(g * 1024 + (j + 1) * 128, 128)],
                rows_v.at[1 - b], sem_in)
        out_d[j] = pltpu.async_copy(rows_v.at[b], out_hbm.at[pos8_v.at[j]],
                                    sem_out)
        idx_d[j] = pltpu.async_copy(dec_v.at[pl.ds(j * 128, 128)],
                                    idx_hbm.at[pos8_v.at[j]], sem_out)
    out_d[7].wait()
    idx_d[7].wait()


def kernel(feats, coords, kernel):
    kmat = kernel
    out72, bin2d = _run_matmul(coords.astype(jnp.int32), feats,
                               kmat.reshape(KK * CIN, COUT))
    binid = bin2d.reshape(PADN)

    acc, cnt = _scatter_kernel(out72, binid)
    pos2d, mtot = _run_prefix(cnt.reshape(256, 128))
    ar = jnp.arange(NBINS, dtype=jnp.int32)[:, None]
    cid = jnp.arange(16, dtype=jnp.int32)[None, :]
    dec = jnp.where(cid == 0, ar // 1024,
                    jnp.where(cid == 2, (ar // 32) % 32,
                              jnp.where(cid == 3, ar % 32, 0)))
    vals, idx16 = _compact_kernel(acc, pos2d, dec)

    total = mtot[0, 0]
    rid = jnp.arange(N, dtype=jnp.int32)[:, None]
    vals_out = jnp.where(rid < total, vals[:N, :COUT], 0.0)
    pad_row = jnp.array([-1, 31, 31, 31], jnp.int32)
    idx_out = jnp.where(rid < total, idx16[:N, :4], pad_row[None, :])
    return vals_out, idx_out


# speedup vs baseline: 1.3445x; 1.3241x over previous
import functools

import jax
import jax.numpy as jnp
from jax import lax
from jax.experimental import pallas as pl
from jax.experimental.pallas import tpu as pltpu
from jax.experimental.pallas import tpu_sc as plsc

N = 50000
CIN = 64
COUT = 64
W = 72
KK = 32
NBINS = 32768
HALF = NBINS // 2
PADN = 51200
TN = 512
PPT = PADN // 16

_I16 = lambda: lax.iota(jnp.int32, 16)


def _mm_body(coords_ref, feats_ref, kflat_ref, out_ref, bin_ref):
    n = pl.program_id(0)
    c1 = coords_ref[:, 1:2]
    f = feats_ref[...].astype(jnp.bfloat16)
    kb = kflat_ref[...].astype(jnp.bfloat16)
    parts = []
    for k in range(KK):
        m = (c1 == k).astype(jnp.bfloat16)
        parts.append(f * m)
    stacked = jnp.concatenate(parts, axis=1)
    acc = jnp.dot(stacked, kb, preferred_element_type=jnp.float32)
    out_ref[...] = jnp.concatenate(
        [acc, jnp.ones((TN, W - COUT), jnp.float32)], axis=1)
    rowid = n * TN + lax.broadcasted_iota(jnp.int32, (TN, 1), 0)
    b = (coords_ref[:, 0:1] * 1024 + coords_ref[:, 2:3] * 32
         + coords_ref[:, 3:4])
    bin_ref[...] = jnp.where(rowid < N, b, NBINS).reshape(1, TN // 128, 128)


def _run_matmul(coords, feats, kflat):
    nlast = (N - 1) // TN

    def _in_map(n):
        return (jnp.minimum(n, nlast), 0)

    return pl.pallas_call(
        _mm_body,
        grid=(PADN // TN,),
        in_specs=[
            pl.BlockSpec((TN, 4), _in_map),
            pl.BlockSpec((TN, CIN), _in_map),
            pl.BlockSpec((KK * CIN, COUT), lambda n: (0, 0)),
        ],
        out_specs=[
            pl.BlockSpec((TN, W), lambda n: (n, 0)),
            pl.BlockSpec((1, TN // 128, 128), lambda n: (n, 0, 0)),
        ],
        out_shape=[
            jax.ShapeDtypeStruct((PADN, W), jnp.float32),
            jax.ShapeDtypeStruct((PADN // TN, TN // 128, 128), jnp.int32),
        ],
    )(coords, feats, kflat)


def _pf_body(cnt_ref, pos_ref, mtot_ref):
    if True:
        occ = (cnt_ref[...] > 0).astype(jnp.float32)
        ir = lax.broadcasted_iota(jnp.int32, (128, 128), 0)
        ic = lax.broadcasted_iota(jnp.int32, (128, 128), 1)
        slt = (ir < ic).astype(jnp.float32)
        ex_row = jnp.dot(occ, slt, preferred_element_type=jnp.float32)
        row_tot = jnp.dot(occ, jnp.ones((128, 1), jnp.float32),
                          preferred_element_type=jnp.float32)
        jr = lax.broadcasted_iota(jnp.int32, (256, 256), 0)
        jc = lax.broadcasted_iota(jnp.int32, (256, 256), 1)
        slt2 = (jc < jr).astype(jnp.float32)
        row_off = jnp.dot(slt2, row_tot, preferred_element_type=jnp.float32)
        posf = row_off + ex_row
        br = lax.broadcasted_iota(jnp.int32, (256, 128), 0)
        bc = lax.broadcasted_iota(jnp.int32, (256, 128), 1)
        trash = PADN + ((br * 128 + bc) & 8191)
        pos_ref[...] = jnp.where(occ > 0, posf.astype(jnp.int32), trash)
        total = jnp.sum(row_tot).astype(jnp.int32)
        mtot_ref[...] = jnp.full((8, 128), total, jnp.int32)


def _run_prefix(cnt2d):
    return pl.pallas_call(
        _pf_body,
        out_shape=[
            jax.ShapeDtypeStruct((256, 128), jnp.int32),
            jax.ShapeDtypeStruct((8, 128), jnp.int32),
        ],
    )(cnt2d)


_MESH = plsc.VectorSubcoreMesh(core_axis_name="c", subcore_axis_name="s")


@functools.partial(
    pl.kernel,
    out_type=(
        jax.ShapeDtypeStruct((NBINS + 1, W), jnp.float32),
        jax.ShapeDtypeStruct((NBINS,), jnp.int32),
    ),
    mesh=_MESH,
    compiler_params=pltpu.CompilerParams(use_tc_tiling_on_sc=False),
    scratch_types=(
        pltpu.VMEM((2, 128, W), jnp.float32),
        pltpu.VMEM((PPT,), jnp.int32),
        pltpu.VMEM((2, 128), jnp.int32),
        pltpu.VMEM((1024,), jnp.int32),
        pltpu.VMEM_SHARED((HALF + 1, W), jnp.float32),
        pltpu.SemaphoreType.DMA,
        pltpu.SemaphoreType.DMA,
    ),
)
def _scatter_kernel(out72_hbm, bid_hbm, acc_hbm, cnt_hbm,
                    rows_v, bid_v, idx_v, c1d_v, acc_sh, sem_in, sem_sc):
    c = lax.axis_index("c")
    s = lax.axis_index("s")
    zf = jnp.zeros((16,), jnp.float32)

    def _zero_rows(i, _):
        for b in range(2):
            for cw in range(4):
                rows_v[b, i, pl.ds(cw * 16, 16)] = zf
            rows_v[b, i, pl.ds(W - 16, 16)] = zf
        return 0

    lax.fori_loop(0, 128, _zero_rows, 0)

    zdescs = [
        pltpu.async_copy(rows_v.at[h % 2],
                         acc_sh.at[pl.ds(s * 1024 + h * 128, 128)], sem_in)
        for h in range(8)
    ]

    @pl.when(s == 0)
    def _():
        pltpu.sync_copy(rows_v.at[0, pl.ds(0, 1)], acc_sh.at[pl.ds(HALF, 1)])

    for d in zdescs:
        d.wait()
    plsc.subcore_barrier()

    base = s * PPT
    nch = PPT // 128
    pltpu.sync_copy(bid_hbm.at[pl.ds(base, PPT)], bid_v)
    in_descs = [None] * nch
    sc_descs = [None] * nch
    in_descs[0] = pltpu.async_copy(out72_hbm.at[pl.ds(base, 128)],
                                   rows_v.at[0], sem_in)
    for ch in range(nch):
        b = ch % 2
        in_descs[ch].wait()
        if ch >= 1:
            sc_descs[ch - 1].wait()
        if ch + 1 < nch:
            in_descs[ch + 1] = pltpu.async_copy(
                out72_hbm.at[pl.ds(base + (ch + 1) * 128, 128)],
                rows_v.at[1 - b], sem_in)

        def _mk_idx(i, _, ch=ch, b=b):
            bd = bid_v[pl.ds(ch * 128 + i * 16, 16)]
            loc = bd - c * HALF
            ok = (loc >= 0) & (loc < HALF)
            loc = jnp.where(ok, loc, HALF)
            idx_v[b, pl.ds(i * 16, 16)] = loc
            return 0

        lax.fori_loop(0, 8, _mk_idx, 0)
        sc_descs[ch] = pltpu.async_copy(rows_v.at[b], acc_sh.at[idx_v.at[b]],
                                        sem_sc, add=True)
    sc_descs[nch - 1].wait()

    plsc.subcore_barrier()

    g = c * 16 + s
    gbase = g * 1024
    lane = _I16()
    wb_in = [None] * 8
    wb_out = [None] * 8
    wb_in[0] = pltpu.async_copy(acc_sh.at[pl.ds(s * 1024, 128)],
                                rows_v.at[0], sem_in)
    for h in range(8):
        b = h % 2
        wb_in[h].wait()
        if h >= 1:
            wb_out[h - 1].wait()
        if h + 1 < 8:
            wb_in[h + 1] = pltpu.async_copy(
                acc_sh.at[pl.ds(s * 1024 + (h + 1) * 128, 128)],
                rows_v.at[1 - b], sem_in)
        wb_out[h] = pltpu.async_copy(
            rows_v.at[b], acc_hbm.at[pl.ds(gbase + h * 128, 128)], sem_sc)

        def _extract(w, _, h=h, b=b):
            cv = jnp.zeros((16,), jnp.float32)
            for r in range(16):
                cnt_r = rows_v[b, w * 16 + r, pl.ds(W - 16, 16)][8]
                cv = jnp.where(lane == r, cnt_r, cv)
            c1d_v[pl.ds(h * 128 + w * 16, 16)] = cv.astype(jnp.int32)
            return 0

        lax.fori_loop(0, 8, _extract, 0)
    wb_out[7].wait()
    pltpu.sync_copy(c1d_v, cnt_hbm.at[pl.ds(gbase, 1024)])


OUTR = PADN + 8192


@functools.partial(
    pl.kernel,
    out_type=(
        jax.ShapeDtypeStruct((OUTR, W), jnp.float32),
        jax.ShapeDtypeStruct((OUTR, 16), jnp.int32),
    ),
    mesh=_MESH,
    compiler_params=pltpu.CompilerParams(use_tc_tiling_on_sc=False),
    scratch_types=(
        pltpu.VMEM((8, 128), jnp.int32),
        pltpu.VMEM((2, 128, W), jnp.float32),
        pltpu.VMEM((1024, 16), jnp.int32),
        pltpu.SemaphoreType.DMA,
        pltpu.SemaphoreType.DMA,
    ),
)
def _compact_kernel(acc_hbm, pos_hbm, dec_hbm, out_hbm, idx_hbm,
                    pos8_v, rows_v, dec_v, sem_in, sem_out):
    c = lax.axis_index("c")
    s = lax.axis_index("s")
    g = c * 16 + s

    d_pos = pltpu.async_copy(pos_hbm.at[pl.ds(g * 8, 8)], pos8_v, sem_in)
    d_dec = pltpu.async_copy(dec_hbm.at[pl.ds(g * 1024, 1024)], dec_v, sem_in)
    d_pos.wait()
    d_dec.wait()

    in_d = [None] * 8
    out_d = [None] * 8
    idx_d = [None] * 8
    in_d[0] = pltpu.async_copy(acc_hbm.at[pl.ds(g * 1024, 128)],
                               rows_v.at[0], sem_in)
    for j in range(8):
        b = j % 2
        in_d[j].wait()
        if j >= 1:
            out_d[j - 1].wait()
            idx_d[j - 1].wait()
        if j + 1 < 8:
            in_d[j + 1] = pltpu.async_copy(
                acc_hbm.at[pl.ds(g * 1024 + (j + 1) * 128, 128)],
                rows_v.at[1 - b], sem_in)
        out_d[j] = pltpu.async_copy(rows_v.at[b], out_hbm.at[pos8_v.at[j]],
                                    sem_out)
        idx_d[j] = pltpu.async_copy(dec_v.at[pl.ds(j * 128, 128)],
                                    idx_hbm.at[pos8_v.at[j]], sem_out)
    out_d[7].wait()
    idx_d[7].wait()


def kernel(feats, coords, kernel):
    kmat = kernel
    out72, bin2d = _run_matmul(coords.astype(jnp.int32), feats,
                               kmat.reshape(KK * CIN, COUT))
    binid = bin2d.reshape(PADN)

    acc, cnt = _scatter_kernel(out72, binid)
    pos2d, mtot = _run_prefix(cnt.reshape(256, 128))
    ar = jnp.arange(NBINS, dtype=jnp.int32)[:, None]
    cid = jnp.arange(16, dtype=jnp.int32)[None, :]
    dec = jnp.where(cid == 0, ar // 1024,
                    jnp.where(cid == 2, (ar // 32) % 32,
                              jnp.where(cid == 3, ar % 32, 0)))
    vals, idx16 = _compact_kernel(acc, pos2d, dec)

    total = mtot[0, 0]
    rid = jnp.arange(N, dtype=jnp.int32)[:, None]
    vals_out = jnp.where(rid < total, vals[:N, :COUT], 0.0)
    pad_row = jnp.array([-1, 31, 31, 31], jnp.int32)
    idx_out = jnp.where(rid < total, idx16[:N, :4], pad_row[None, :])
    return vals_out, idx_out
